# baseline (speedup 1.0000x reference)
import jax
import jax.numpy as jnp
from jax import lax
from jax.experimental import pallas as pl
from jax.experimental.pallas import tpu as pltpu
from jax.experimental.pallas import tpu_sc as plsc

N_NODES = 10000
N_EDGES = 160000
H = 256
HALF = 128
ROW = 128
N_ROWS = N_EDGES // ROW
N_SUB = 16
STRIPE = N_NODES // N_SUB
NBUF = 2


def _sc_segment_sum(ea, col, zeros):
    mesh = plsc.VectorSubcoreMesh(core_axis_name="c", subcore_axis_name="s")

    def body(ea_hbm, col_hbm, z_hbm, agg0_hbm, agg1_hbm, idx_v, rows_v, accum,
             sem, sem2):
        c = lax.axis_index("c")
        s = lax.axis_index("s")
        pltpu.sync_copy(z_hbm, accum.at[pl.ds(s * STRIPE, STRIPE)])
        plsc.subcore_barrier()

        col0 = c * HALF
        base = N_ROWS // N_SUB
        rem = N_ROWS - base * N_SUB
        nch = jnp.where(s < rem, base + 1, base)

        def start(k, b):
            row = s + N_SUB * k
            pltpu.async_copy(col_hbm.at[:, pl.ds(row * ROW, ROW)], idx_v.at[b],
                             sem)
            pltpu.async_copy(ea_hbm.at[pl.ds(row * ROW, ROW),
                                       pl.ds(col0, HALF)],
                             rows_v.at[b], sem)

        def wait(b):
            pltpu.make_async_copy(col_hbm.at[:, pl.ds(0, ROW)], idx_v.at[b],
                                  sem).wait()
            pltpu.make_async_copy(ea_hbm.at[pl.ds(0, ROW), pl.ds(col0, HALF)],
                                  rows_v.at[b], sem).wait()

        def drain_scatter():
            pltpu.make_async_copy(
                ea_hbm.at[pl.ds(0, ROW), pl.ds(col0, HALF)],
                accum.at[pl.ds(0, ROW)], sem2).wait()

        start(0, 0)
        start(1, 1)

        @pl.loop(0, nch)
        def _(kk):
            b = kk % 2
            wait(b)
            pltpu.async_copy(rows_v.at[b], accum.at[idx_v.at[b, 1]], sem2,
                             add=True)

            @pl.when(kk >= 1)
            def _():
                drain_scatter()

                @pl.when(kk + 1 < nch)
                def _():
                    start(kk + 1, 1 - b)

        drain_scatter()

        plsc.subcore_barrier()
        a0 = (s * STRIPE) // 8 * 8
        a1 = jnp.where(s == N_SUB - 1, N_NODES, ((s + 1) * STRIPE) // 8 * 8)
        size = a1 - a0

        def writeout(dst_hbm, n):
            pltpu.sync_copy(accum.at[pl.ds(a0, n)], dst_hbm.at[pl.ds(a0, n)])

        for n in (624, 632):
            @pl.when((size == n) & (c == 0))
            def _():
                writeout(agg0_hbm, n)

            @pl.when((size == n) & (c == 1))
            def _():
                writeout(agg1_hbm, n)

    f = pl.kernel(
        body,
        out_type=[jax.ShapeDtypeStruct((N_NODES, HALF), jnp.float32),
                  jax.ShapeDtypeStruct((N_NODES, HALF), jnp.float32)],
        mesh=mesh,
        scratch_types=[
            pltpu.VMEM((NBUF, 2, ROW), jnp.int32),
            pltpu.VMEM((NBUF, ROW, HALF), jnp.float32),
            pltpu.VMEM_SHARED((N_NODES, HALF), jnp.float32),
            pltpu.SemaphoreType.DMA,
            pltpu.SemaphoreType.DMA,
        ],
    )
    return f(ea, col, zeros)


BLK = 1000


def _mlp_partial(x, w1, b1):
    def body(x_ref, w1_ref, b1_ref, o_ref):
        o_ref[...] = (jnp.dot(x_ref[...], w1_ref[:H],
                              preferred_element_type=jnp.float32)
                      + b1_ref[...])

    return pl.pallas_call(
        body,
        grid=(N_NODES // BLK,),
        in_specs=[
            pl.BlockSpec((BLK, H), lambda i: (i, 0)),
            pl.BlockSpec((2 * H, H), lambda i: (0, 0)),
            pl.BlockSpec((1, H), lambda i: (0, 0)),
        ],
        out_specs=pl.BlockSpec((BLK, H), lambda i: (i, 0)),
        out_shape=jax.ShapeDtypeStruct((N_NODES, H), jnp.float32),
    )(x, w1, b1)


def _mlp_final(partial, agg0, agg1, w1, w2, b2):
    def body(p_ref, a0_ref, a1_ref, w1_ref, w2_ref, b2_ref, o_ref):
        h = p_ref[...]
        h += jnp.dot(a0_ref[...], w1_ref[H:H + HALF],
                     preferred_element_type=jnp.float32)
        h += jnp.dot(a1_ref[...], w1_ref[H + HALF:],
                     preferred_element_type=jnp.float32)
        h = jnp.maximum(h, 0.0)
        o_ref[...] = (jnp.dot(h, w2_ref[...], preferred_element_type=jnp.float32)
                      + b2_ref[...])

    return pl.pallas_call(
        body,
        grid=(N_NODES // BLK,),
        in_specs=[
            pl.BlockSpec((BLK, H), lambda i: (i, 0)),
            pl.BlockSpec((BLK, HALF), lambda i: (i, 0)),
            pl.BlockSpec((BLK, HALF), lambda i: (i, 0)),
            pl.BlockSpec((2 * H, H), lambda i: (0, 0)),
            pl.BlockSpec((H, H), lambda i: (0, 0)),
            pl.BlockSpec((1, H), lambda i: (0, 0)),
        ],
        out_specs=pl.BlockSpec((BLK, H), lambda i: (i, 0)),
        out_shape=jax.ShapeDtypeStruct((N_NODES, H), jnp.float32),
    )(partial, agg0, agg1, w1, w2, b2)


def kernel(x, edge_index, edge_attr, u, batch, W1, b1, W2, b2):
    if edge_index.dtype != jnp.int32:
        edge_index = edge_index.astype(jnp.int32)
    zeros = jnp.zeros((STRIPE, HALF), jnp.float32)
    agg0, agg1 = _sc_segment_sum(edge_attr, edge_index, zeros)
    partial = _mlp_partial(x, W1, b1.reshape(1, H))
    return _mlp_final(partial, agg0, agg1, W1, W2, b2.reshape(1, H))


# speedup vs baseline: 1.1724x; 1.1724x over previous
import jax
import jax.numpy as jnp
from jax import lax
from jax.experimental import pallas as pl
from jax.experimental.pallas import tpu as pltpu
from jax.experimental.pallas import tpu_sc as plsc

N_NODES = 10000
N_EDGES = 160000
H = 256
HALF = 128
ROW = 128
N_ROWS = N_EDGES // ROW
N_SUB = 16
STRIPE = N_NODES // N_SUB
NBUF = 2


def _sc_segment_sum(ea, col, zeros):
    mesh = plsc.VectorSubcoreMesh(core_axis_name="c", subcore_axis_name="s")

    def body(ea_hbm, col_hbm, z_hbm, agg0_hbm, agg1_hbm, idx_v, rows_v, accum,
             sem):
        c = lax.axis_index("c")
        s = lax.axis_index("s")
        pltpu.sync_copy(z_hbm, accum.at[pl.ds(s * STRIPE, STRIPE)])
        plsc.subcore_barrier()

        col0 = c * HALF
        base = N_ROWS // N_SUB
        rem = N_ROWS - base * N_SUB
        nch = jnp.where(s < rem, base + 1, base)

        def start(k, b):
            row = s + N_SUB * k
            pltpu.async_copy(col_hbm.at[:, pl.ds(row * ROW, ROW)], idx_v.at[b],
                             sem)
            pltpu.async_copy(ea_hbm.at[pl.ds(row * ROW, ROW),
                                       pl.ds(col0, HALF)],
                             rows_v.at[b], sem)

        def wait(b):
            pltpu.make_async_copy(col_hbm.at[:, pl.ds(0, ROW)], idx_v.at[b],
                                  sem).wait()
            pltpu.make_async_copy(ea_hbm.at[pl.ds(0, ROW), pl.ds(col0, HALF)],
                                  rows_v.at[b], sem).wait()

        start(0, 0)
        start(1, 1)

        @pl.loop(0, (base + NBUF - 1) // NBUF * NBUF + NBUF, step=NBUF)
        def _(k):
            for b in range(NBUF):
                kk = k + b

                @pl.when(kk < nch)
                def _():
                    wait(b)
                    pltpu.sync_copy(rows_v.at[b], accum.at[idx_v.at[b, 1]],
                                    add=True)

                    @pl.when(kk + NBUF < nch)
                    def _():
                        start(kk + NBUF, b)

        plsc.subcore_barrier()
        a0 = (s * STRIPE) // 8 * 8
        a1 = jnp.where(s == N_SUB - 1, N_NODES, ((s + 1) * STRIPE) // 8 * 8)
        size = a1 - a0

        def writeout(dst_hbm, n):
            pltpu.sync_copy(accum.at[pl.ds(a0, n)], dst_hbm.at[pl.ds(a0, n)])

        for n in (624, 632):
            @pl.when((size == n) & (c == 0))
            def _():
                writeout(agg0_hbm, n)

            @pl.when((size == n) & (c == 1))
            def _():
                writeout(agg1_hbm, n)

    f = pl.kernel(
        body,
        out_type=[jax.ShapeDtypeStruct((N_NODES, HALF), jnp.float32),
                  jax.ShapeDtypeStruct((N_NODES, HALF), jnp.float32)],
        mesh=mesh,
        scratch_types=[
            pltpu.VMEM((NBUF, 2, ROW), jnp.int32),
            pltpu.VMEM((NBUF, ROW, HALF), jnp.float32),
            pltpu.VMEM_SHARED((N_NODES, HALF), jnp.float32),
            pltpu.SemaphoreType.DMA,
        ],
    )
    return f(ea, col, zeros)


BLK = 2000


def _mlp_partial(x, w1, b1):
    def body(x_ref, w1_ref, b1_ref, o_ref):
        o_ref[...] = (jnp.dot(x_ref[...], w1_ref[:H],
                              preferred_element_type=jnp.float32)
                      + b1_ref[...])

    return pl.pallas_call(
        body,
        grid=(N_NODES // BLK,),
        in_specs=[
            pl.BlockSpec((BLK, H), lambda i: (i, 0)),
            pl.BlockSpec((2 * H, H), lambda i: (0, 0)),
            pl.BlockSpec((1, H), lambda i: (0, 0)),
        ],
        out_specs=pl.BlockSpec((BLK, H), lambda i: (i, 0)),
        out_shape=jax.ShapeDtypeStruct((N_NODES, H), jnp.float32),
    )(x, w1, b1)


def _mlp_final(partial, agg0, agg1, w1, w2, b2):
    def body(p_ref, a0_ref, a1_ref, w1_ref, w2_ref, b2_ref, o_ref):
        h = p_ref[...]
        h += jnp.dot(a0_ref[...], w1_ref[H:H + HALF],
                     preferred_element_type=jnp.float32)
        h += jnp.dot(a1_ref[...], w1_ref[H + HALF:],
                     preferred_element_type=jnp.float32)
        h = jnp.maximum(h, 0.0)
        o_ref[...] = (jnp.dot(h, w2_ref[...], preferred_element_type=jnp.float32)
                      + b2_ref[...])

    return pl.pallas_call(
        body,
        grid=(N_NODES // BLK,),
        in_specs=[
            pl.BlockSpec((BLK, H), lambda i: (i, 0)),
            pl.BlockSpec((BLK, HALF), lambda i: (i, 0)),
            pl.BlockSpec((BLK, HALF), lambda i: (i, 0)),
            pl.BlockSpec((2 * H, H), lambda i: (0, 0)),
            pl.BlockSpec((H, H), lambda i: (0, 0)),
            pl.BlockSpec((1, H), lambda i: (0, 0)),
        ],
        out_specs=pl.BlockSpec((BLK, H), lambda i: (i, 0)),
        out_shape=jax.ShapeDtypeStruct((N_NODES, H), jnp.float32),
    )(partial, agg0, agg1, w1, w2, b2)


def kernel(x, edge_index, edge_attr, u, batch, W1, b1, W2, b2):
    if edge_index.dtype != jnp.int32:
        edge_index = edge_index.astype(jnp.int32)
    zeros = jnp.zeros((STRIPE, HALF), jnp.float32)
    agg0, agg1 = _sc_segment_sum(edge_attr, edge_index, zeros)
    partial = _mlp_partial(x, W1, b1.reshape(1, H))
    return _mlp_final(partial, agg0, agg1, W1, W2, b2.reshape(1, H))


# speedup vs baseline: 1.2584x; 1.0734x over previous
import jax
import jax.numpy as jnp
from jax import lax
from jax.experimental import pallas as pl
from jax.experimental.pallas import tpu as pltpu
from jax.experimental.pallas import tpu_sc as plsc

N_NODES = 10000
N_EDGES = 160000
H = 256
HALF = 128
ROW = 128
N_ROWS = N_EDGES // ROW
N_SUB = 16
STRIPE = N_NODES // N_SUB
NBUF = 3


def _sc_segment_sum(ea, col, zeros):
    mesh = plsc.VectorSubcoreMesh(core_axis_name="c", subcore_axis_name="s")

    def body(ea_hbm, col_hbm, z_hbm, agg0_hbm, agg1_hbm, idx_v, rows_v, accum,
             sem):
        c = lax.axis_index("c")
        s = lax.axis_index("s")
        col0 = c * HALF
        base = N_ROWS // N_SUB
        rem = N_ROWS - base * N_SUB
        nch = jnp.where(s < rem, base + 1, base)

        def start(k, b):
            row = s + N_SUB * k
            pltpu.async_copy(col_hbm.at[:, pl.ds(row * ROW, ROW)], idx_v.at[b],
                             sem)
            pltpu.async_copy(ea_hbm.at[pl.ds(row * ROW, ROW),
                                       pl.ds(col0, HALF)],
                             rows_v.at[b], sem)

        def wait(b):
            pltpu.make_async_copy(col_hbm.at[:, pl.ds(0, ROW)], idx_v.at[b],
                                  sem).wait()
            pltpu.make_async_copy(ea_hbm.at[pl.ds(0, ROW), pl.ds(col0, HALF)],
                                  rows_v.at[b], sem).wait()

        for b in range(NBUF):
            start(b, b)
        pltpu.sync_copy(z_hbm, accum.at[pl.ds(s * STRIPE, STRIPE)])
        plsc.subcore_barrier()

        @pl.loop(0, (base + NBUF - 1) // NBUF * NBUF + NBUF, step=NBUF)
        def _(k):
            for b in range(NBUF):
                kk = k + b

                @pl.when(kk < nch)
                def _():
                    wait(b)
                    pltpu.sync_copy(rows_v.at[b], accum.at[idx_v.at[b, 1]],
                                    add=True)

                    @pl.when(kk + NBUF < nch)
                    def _():
                        start(kk + NBUF, b)

        plsc.subcore_barrier()
        a0 = (s * STRIPE) // 8 * 8
        a1 = jnp.where(s == N_SUB - 1, N_NODES, ((s + 1) * STRIPE) // 8 * 8)
        size = a1 - a0

        def writeout(dst_hbm, n):
            pltpu.sync_copy(accum.at[pl.ds(a0, n)], dst_hbm.at[pl.ds(a0, n)])

        for n in (624, 632):
            @pl.when((size == n) & (c == 0))
            def _():
                writeout(agg0_hbm, n)

            @pl.when((size == n) & (c == 1))
            def _():
                writeout(agg1_hbm, n)

    f = pl.kernel(
        body,
        out_type=[jax.ShapeDtypeStruct((N_NODES, HALF), jnp.float32),
                  jax.ShapeDtypeStruct((N_NODES, HALF), jnp.float32)],
        mesh=mesh,
        scratch_types=[
            pltpu.VMEM((NBUF, 2, ROW), jnp.int32),
            pltpu.VMEM((NBUF, ROW, HALF), jnp.float32),
            pltpu.VMEM_SHARED((N_NODES, HALF), jnp.float32),
            pltpu.SemaphoreType.DMA,
        ],
    )
    return f(ea, col, zeros)


BLK = 2000


def _mlp_partial(x, w1, b1):
    def body(x_ref, w1_ref, b1_ref, o_ref):
        o_ref[...] = (jnp.dot(x_ref[...], w1_ref[:H],
                              preferred_element_type=jnp.float32)
                      + b1_ref[...])

    return pl.pallas_call(
        body,
        grid=(N_NODES // BLK,),
        in_specs=[
            pl.BlockSpec((BLK, H), lambda i: (i, 0)),
            pl.BlockSpec((2 * H, H), lambda i: (0, 0)),
            pl.BlockSpec((1, H), lambda i: (0, 0)),
        ],
        out_specs=pl.BlockSpec((BLK, H), lambda i: (i, 0)),
        out_shape=jax.ShapeDtypeStruct((N_NODES, H), jnp.float32),
    )(x, w1, b1)


def _mlp_final(partial, agg0, agg1, w1, w2, b2):
    def body(p_ref, a0_ref, a1_ref, w1_ref, w2_ref, b2_ref, o_ref):
        h = p_ref[...]
        h += jnp.dot(a0_ref[...], w1_ref[H:H + HALF],
                     preferred_element_type=jnp.float32)
        h += jnp.dot(a1_ref[...], w1_ref[H + HALF:],
                     preferred_element_type=jnp.float32)
        h = jnp.maximum(h, 0.0)
        o_ref[...] = (jnp.dot(h, w2_ref[...], preferred_element_type=jnp.float32)
                      + b2_ref[...])

    return pl.pallas_call(
        body,
        grid=(N_NODES // BLK,),
        in_specs=[
            pl.BlockSpec((BLK, H), lambda i: (i, 0)),
            pl.BlockSpec((BLK, HALF), lambda i: (i, 0)),
            pl.BlockSpec((BLK, HALF), lambda i: (i, 0)),
            pl.BlockSpec((2 * H, H), lambda i: (0, 0)),
            pl.BlockSpec((H, H), lambda i: (0, 0)),
            pl.BlockSpec((1, H), lambda i: (0, 0)),
        ],
        out_specs=pl.BlockSpec((BLK, H), lambda i: (i, 0)),
        out_shape=jax.ShapeDtypeStruct((N_NODES, H), jnp.float32),
    )(partial, agg0, agg1, w1, w2, b2)


def kernel(x, edge_index, edge_attr, u, batch, W1, b1, W2, b2):
    if edge_index.dtype != jnp.int32:
        edge_index = edge_index.astype(jnp.int32)
    zeros = jnp.zeros((STRIPE, HALF), jnp.float32)
    agg0, agg1 = _sc_segment_sum(edge_attr, edge_index, zeros)
    partial = _mlp_partial(x, W1, b1.reshape(1, H))
    return _mlp_final(partial, agg0, agg1, W1, W2, b2.reshape(1, H))


# speedup vs baseline: 1.2860x; 1.0220x over previous
import jax
import jax.numpy as jnp
from jax import lax
from jax.experimental import pallas as pl
from jax.experimental.pallas import tpu as pltpu
from jax.experimental.pallas import tpu_sc as plsc

N_NODES = 10000
N_EDGES = 160000
H = 256
HALF = 128
ROW = 128
N_ROWS = N_EDGES // ROW
N_SUB = 16
STRIPE = N_NODES // N_SUB
NBUF = 3


def _sc_segment_sum(ea, col, zeros):
    mesh = plsc.VectorSubcoreMesh(core_axis_name="c", subcore_axis_name="s")

    def body(ea_hbm, col_hbm, z_hbm, agg0_hbm, agg1_hbm, idx_v, rows_v, accum,
             sem):
        c = lax.axis_index("c")
        s = lax.axis_index("s")
        col0 = c * HALF
        base = N_ROWS // N_SUB
        rem = N_ROWS - base * N_SUB
        nch = jnp.where(s < rem, base + 1, base)

        def start(k, b):
            row = s + N_SUB * k
            pltpu.async_copy(col_hbm.at[:, pl.ds(row * ROW, ROW)], idx_v.at[b],
                             sem)
            pltpu.async_copy(ea_hbm.at[pl.ds(row * ROW, ROW),
                                       pl.ds(col0, HALF)],
                             rows_v.at[b], sem)

        def wait(b):
            pltpu.make_async_copy(col_hbm.at[:, pl.ds(0, ROW)], idx_v.at[b],
                                  sem).wait()
            pltpu.make_async_copy(ea_hbm.at[pl.ds(0, ROW), pl.ds(col0, HALF)],
                                  rows_v.at[b], sem).wait()

        for b in range(NBUF):
            start(b, b)
        pltpu.sync_copy(z_hbm, accum.at[pl.ds(s * STRIPE, STRIPE)])
        plsc.subcore_barrier()

        @pl.loop(0, (base + NBUF - 1) // NBUF * NBUF + NBUF, step=NBUF)
        def _(k):
            for b in range(NBUF):
                kk = k + b

                @pl.when(kk < nch)
                def _():
                    wait(b)
                    pltpu.sync_copy(rows_v.at[b], accum.at[idx_v.at[b, 1]],
                                    add=True)

                    @pl.when(kk + NBUF < nch)
                    def _():
                        start(kk + NBUF, b)

        plsc.subcore_barrier()
        a0 = (s * STRIPE) // 8 * 8
        a1 = jnp.where(s == N_SUB - 1, N_NODES, ((s + 1) * STRIPE) // 8 * 8)
        size = a1 - a0

        def writeout(dst_hbm, n):
            pltpu.sync_copy(accum.at[pl.ds(a0, n)], dst_hbm.at[pl.ds(a0, n)])

        for n in (624, 632):
            @pl.when((size == n) & (c == 0))
            def _():
                writeout(agg0_hbm, n)

            @pl.when((size == n) & (c == 1))
            def _():
                writeout(agg1_hbm, n)

    f = pl.kernel(
        body,
        out_type=[jax.ShapeDtypeStruct((N_NODES, HALF), jnp.float32),
                  jax.ShapeDtypeStruct((N_NODES, HALF), jnp.float32)],
        mesh=mesh,
        scratch_types=[
            pltpu.VMEM((NBUF, 2, ROW), jnp.int32),
            pltpu.VMEM((NBUF, ROW, HALF), jnp.float32),
            pltpu.VMEM_SHARED((N_NODES, HALF), jnp.float32),
            pltpu.SemaphoreType.DMA,
        ],
    )
    return f(ea, col, zeros)


BLK = 2000


def _mlp_partial(x, w1, b1):
    def body(x_ref, w1_ref, b1_ref, o_ref):
        o_ref[...] = (jnp.dot(x_ref[...], w1_ref[:H],
                              preferred_element_type=jnp.float32)
                      + b1_ref[...]).astype(jnp.bfloat16)

    return pl.pallas_call(
        body,
        grid=(N_NODES // BLK,),
        in_specs=[
            pl.BlockSpec((BLK, H), lambda i: (i, 0)),
            pl.BlockSpec((2 * H, H), lambda i: (0, 0)),
            pl.BlockSpec((1, H), lambda i: (0, 0)),
        ],
        out_specs=pl.BlockSpec((BLK, H), lambda i: (i, 0)),
        out_shape=jax.ShapeDtypeStruct((N_NODES, H), jnp.bfloat16),
    )(x, w1, b1)


def _mlp_final(partial, agg0, agg1, w1, w2, b2):
    def body(p_ref, a0_ref, a1_ref, w1_ref, w2_ref, b2_ref, o_ref):
        h = p_ref[...].astype(jnp.float32)
        h += jnp.dot(a0_ref[...], w1_ref[H:H + HALF],
                     preferred_element_type=jnp.float32)
        h += jnp.dot(a1_ref[...], w1_ref[H + HALF:],
                     preferred_element_type=jnp.float32)
        h = jnp.maximum(h, 0.0)
        o_ref[...] = (jnp.dot(h, w2_ref[...], preferred_element_type=jnp.float32)
                      + b2_ref[...])

    return pl.pallas_call(
        body,
        grid=(N_NODES // BLK,),
        in_specs=[
            pl.BlockSpec((BLK, H), lambda i: (i, 0)),
            pl.BlockSpec((BLK, HALF), lambda i: (i, 0)),
            pl.BlockSpec((BLK, HALF), lambda i: (i, 0)),
            pl.BlockSpec((2 * H, H), lambda i: (0, 0)),
            pl.BlockSpec((H, H), lambda i: (0, 0)),
            pl.BlockSpec((1, H), lambda i: (0, 0)),
        ],
        out_specs=pl.BlockSpec((BLK, H), lambda i: (i, 0)),
        out_shape=jax.ShapeDtypeStruct((N_NODES, H), jnp.float32),
    )(partial, agg0, agg1, w1, w2, b2)


def kernel(x, edge_index, edge_attr, u, batch, W1, b1, W2, b2):
    if edge_index.dtype != jnp.int32:
        edge_index = edge_index.astype(jnp.int32)
    zeros = jnp.zeros((STRIPE, HALF), jnp.float32)
    agg0, agg1 = _sc_segment_sum(edge_attr, edge_index, zeros)
    partial = _mlp_partial(x, W1, b1.reshape(1, H))
    return _mlp_final(partial, agg0, agg1, W1, W2, b2.reshape(1, H))
